# baseline (speedup 1.0000x reference)
import functools

import jax
import jax.numpy as jnp
import numpy as np
from jax import lax
from jax.experimental import pallas as pl
from jax.experimental.pallas import tpu as pltpu
from jax.experimental.pallas import tpu_sc as plsc

_B = 16384
_D = 512
_SUB = 128
_N = _D // _SUB
_L = 1000
_NC, _NS = 2, 16
_NW = _NC * _NS
_RW = _B // _NW
_CE = 4
_KE = _RW // _CE
_CL = 4
_KL = _RW // _CL


def _tf2x32(k1, k2, x1, x2):
    def rotl(x, d):
        return (x << np.uint32(d)) | (x >> np.uint32(32 - d))

    def rnds(x0, x1v, rots):
        for r in rots:
            x0 = x0 + x1v
            x1v = rotl(x1v, r)
            x1v = x0 ^ x1v
        return x0, x1v

    r0, r1 = (13, 15, 26, 6), (17, 29, 16, 24)
    ks2 = k1 ^ k2 ^ np.uint32(0x1BD11BDA)
    x0, x1v = x1 + k1, x2 + k2
    x0, x1v = rnds(x0, x1v, r0)
    x0, x1v = x0 + k2, x1v + ks2 + np.uint32(1)
    x0, x1v = rnds(x0, x1v, r1)
    x0, x1v = x0 + ks2, x1v + k1 + np.uint32(2)
    x0, x1v = rnds(x0, x1v, r0)
    x0, x1v = x0 + k1, x1v + k2 + np.uint32(3)
    x0, x1v = rnds(x0, x1v, r1)
    x0, x1v = x0 + k2, x1v + ks2 + np.uint32(4)
    x0, x1v = rnds(x0, x1v, r0)
    return x0 + ks2, x1v + k1 + np.uint32(5)


def _np_fold_in(key, data):
    a, b = _tf2x32(key[0], key[1], np.uint32([0]), np.uint32([data]))
    return np.array([a[0], b[0]], np.uint32)


def _np_random_bits(key, n):
    b1, b2 = _tf2x32(key[0], key[1], np.zeros(n, np.uint32), np.arange(n, dtype=np.uint32))
    return b1 ^ b2


def _np_split(key):
    b1, b2 = _tf2x32(key[0], key[1], np.uint32([0, 0]), np.uint32([0, 1]))
    return (np.array([b1[0], b2[0]], np.uint32), np.array([b1[1], b2[1]], np.uint32))


def _np_uniform01(key, n):
    bits = _np_random_bits(key, n)
    fb = (bits >> np.uint32(9)) | np.uint32(0x3F800000)
    return fb.view(np.float32) - np.float32(1.0)


def _np_permutation(key, n):
    x = np.arange(n, dtype=np.int64)
    exponent = 3
    num_rounds = int(np.ceil(exponent * np.log(max(1, n)) / np.log(np.iinfo(np.uint32).max)))
    for _ in range(num_rounds):
        key, subkey = _np_split(key)
        sort_keys = _np_random_bits(subkey, n)
        x = x[np.argsort(sort_keys, kind="stable")]
    return x


def _build_consts():
    key = np.array([0, 42], np.uint32)
    dec = _np_uniform01(_np_fold_in(key, 0), _B) < 0.5
    perms = [np.arange(_B, dtype=np.int64)]
    for i in range(1, _N):
        perms.append(_np_permutation(_np_fold_in(key, i), _B))
    ar = np.arange(_B, dtype=np.int64)

    srcs = np.stack([np.where(dec if j else np.zeros(_B, bool), perms[j], ar)
                     for j in range(_N)], axis=1)
    eidx = np.ascontiguousarray(
        srcs.reshape(_NW, _KE, _CE * _N).astype(np.int32))
    lidx = np.ascontiguousarray(
        srcs.reshape(_NW, _KL, _CL * _N).astype(np.int32))

    sa = np.ascontiguousarray(np.repeat((~dec).astype(np.float32), 16).reshape(_NW, _RW, 16))
    sb = np.ascontiguousarray(np.repeat(dec.astype(np.float32) * 0.25, 16).reshape(_NW, _RW, 16))
    return eidx, lidx, sa, sb


_CONSTS = _build_consts()


def _body(embed_h, label_h, eidx_h, lidx_h, sa_h, sb_h,
          out_e, out_l, eidxv, lidxv, sav, sbv, egb, eob, lgb, lob,
          eg0, eg1, ew0, ew1, lg0, lg1, lw0, lw1):
    w = lax.axis_index("c") * _NS + lax.axis_index("s")
    wbase = w * _RW
    egsems = (eg0, eg1)
    ewsems = (ew0, ew1)
    lgsems = (lg0, lg1)
    lwsems = (lw0, lw1)

    pltpu.sync_copy(eidx_h.at[w], eidxv)
    pltpu.sync_copy(lidx_h.at[w], lidxv)
    pltpu.sync_copy(sa_h.at[w], sav)
    pltpu.sync_copy(sb_h.at[w], sbv)

    def eg_issue(k, slot):
        pltpu.async_copy(embed_h.at[eidxv.at[k]], egb.at[slot], egsems[slot])

    def eg_wait(slot):
        pltpu.make_async_copy(embed_h.at[eidxv.at[0]], egb.at[slot],
                              egsems[slot]).wait()

    def ew_issue(k, slot):
        pltpu.async_copy(eob.at[slot], out_e.at[pl.ds(wbase + k * _CE, _CE)],
                         ewsems[slot])

    def ew_wait(slot):
        pltpu.make_async_copy(eob.at[slot], out_e.at[pl.ds(0, _CE)],
                              ewsems[slot]).wait()

    def e_assemble(slot):
        for r in range(_CE):
            for j in range(_N):
                for c in range(_SUB // 16):
                    o = j * _SUB + c * 16
                    eob[slot, r, pl.ds(o, 16)] = egb[slot, _N * r + j, pl.ds(o, 16)]

    eg_issue(0, 0)
    eg_issue(1, 1)

    def ebody(i2, _):
        a = 2 * i2
        for slot in range(2):
            k = a + slot
            eg_wait(slot)

            @pl.when(i2 > 0)
            def _drain(slot=slot):
                ew_wait(slot)

            e_assemble(slot)
            ew_issue(k, slot)

            @pl.when(k + 2 < _KE)
            def _pref(k=k, slot=slot):
                eg_issue(k + 2, slot)

        return _

    lax.fori_loop(0, _KE // 2, ebody, None)
    ew_wait(0)
    ew_wait(1)

    def lg_issue(k, slot):
        pltpu.async_copy(label_h.at[lidxv.at[k]], lgb.at[slot], lgsems[slot])

    def lg_wait(slot):
        pltpu.make_async_copy(label_h.at[lidxv.at[0]], lgb.at[slot],
                              lgsems[slot]).wait()

    def lw_issue(k, slot):
        pltpu.async_copy(lob.at[slot], out_l.at[pl.ds(wbase + k * _CL, _CL)],
                         lwsems[slot])

    def lw_wait(slot):
        pltpu.make_async_copy(lob.at[slot], out_l.at[pl.ds(0, _CL)],
                              lwsems[slot]).wait()

    def l_assemble(slot, k):
        def rowb(r, _):
            lr = k * _CL + r
            sa = sav[lr]
            sb = sbv[lr]
            for c in range(62):
                o = c * 16
                lob[slot, r, pl.ds(o, 16)] = lgb[slot, _N * r, pl.ds(o, 16)] * sa
            lob[slot, r, pl.ds(984, 16)] = lgb[slot, _N * r, pl.ds(984, 16)] * sa
            for j in range(1, 5):
                g = _N * r + (j - 1)
                d = j * _L
                for c in range(62):
                    o = c * 16
                    lob[slot, r, pl.ds(d + o, 16)] = lgb[slot, g, pl.ds(o, 16)] * sb
                lob[slot, r, pl.ds(d + 984, 16)] = lgb[slot, g, pl.ds(984, 16)] * sb
            return _

        lax.fori_loop(0, _CL, rowb, None)

    lg_issue(0, 0)
    lg_issue(1, 1)

    def lbody(i2, _):
        a = 2 * i2
        for slot in range(2):
            k = a + slot
            lg_wait(slot)

            @pl.when(i2 > 0)
            def _drain(slot=slot):
                lw_wait(slot)

            l_assemble(slot, k)
            lw_issue(k, slot)

            @pl.when(k + 2 < _KL)
            def _pref(k=k, slot=slot):
                lg_issue(k + 2, slot)

        return _

    lax.fori_loop(0, _KL // 2, lbody, None)
    lw_wait(0)
    lw_wait(1)


@functools.cache
def _sc_call():
    return pl.kernel(
        _body,
        out_type=(
            jax.ShapeDtypeStruct((_B, _D), jnp.float32),
            jax.ShapeDtypeStruct((_B, 5 * _L), jnp.float32),
        ),
        mesh=plsc.VectorSubcoreMesh(
            core_axis_name="c", subcore_axis_name="s", num_cores=_NC, num_subcores=_NS
        ),
        compiler_params=pltpu.CompilerParams(use_tc_tiling_on_sc=False),
        scratch_types=[
            pltpu.VMEM((_KE, _CE * _N), jnp.int32),
            pltpu.VMEM((_KL, _CL * _N), jnp.int32),
            pltpu.VMEM((_RW, 16), jnp.float32),
            pltpu.VMEM((_RW, 16), jnp.float32),
            pltpu.VMEM((2, _CE * _N, _D), jnp.float32),
            pltpu.VMEM((2, _CE, _D), jnp.float32),
            pltpu.VMEM((2, _CL * _N, _L), jnp.float32),
            pltpu.VMEM((2, _CL, 5 * _L), jnp.float32),
        ] + [pltpu.SemaphoreType.DMA] * 8,
    )


def kernel(embed, onehot_label):
    consts = [jnp.asarray(c) for c in _CONSTS]
    return _sc_call()(embed, onehot_label, *consts)


# speedup vs baseline: 1.5935x; 1.5935x over previous
import functools

import jax
import jax.numpy as jnp
import numpy as np
from jax import lax
from jax.experimental import pallas as pl
from jax.experimental.pallas import tpu as pltpu
from jax.experimental.pallas import tpu_sc as plsc

_B = 16384
_D = 512
_SUB = 128
_N = _D // _SUB
_L = 1000
_NC, _NS = 2, 16
_NW = _NC * _NS
_CBE = 128
_KE2 = _B // (_NW * _CBE)
_CBL = 16
_KC = 16
_KQ = 68
_KZ = 80


def _tf2x32(k1, k2, x1, x2):
    def rotl(x, d):
        return (x << np.uint32(d)) | (x >> np.uint32(32 - d))

    def rnds(x0, x1v, rots):
        for r in rots:
            x0 = x0 + x1v
            x1v = rotl(x1v, r)
            x1v = x0 ^ x1v
        return x0, x1v

    r0, r1 = (13, 15, 26, 6), (17, 29, 16, 24)
    ks2 = k1 ^ k2 ^ np.uint32(0x1BD11BDA)
    x0, x1v = x1 + k1, x2 + k2
    x0, x1v = rnds(x0, x1v, r0)
    x0, x1v = x0 + k2, x1v + ks2 + np.uint32(1)
    x0, x1v = rnds(x0, x1v, r1)
    x0, x1v = x0 + ks2, x1v + k1 + np.uint32(2)
    x0, x1v = rnds(x0, x1v, r0)
    x0, x1v = x0 + k1, x1v + k2 + np.uint32(3)
    x0, x1v = rnds(x0, x1v, r1)
    x0, x1v = x0 + k2, x1v + ks2 + np.uint32(4)
    x0, x1v = rnds(x0, x1v, r0)
    return x0 + ks2, x1v + k1 + np.uint32(5)


def _np_fold_in(key, data):
    a, b = _tf2x32(key[0], key[1], np.uint32([0]), np.uint32([data]))
    return np.array([a[0], b[0]], np.uint32)


def _np_random_bits(key, n):
    b1, b2 = _tf2x32(key[0], key[1], np.zeros(n, np.uint32), np.arange(n, dtype=np.uint32))
    return b1 ^ b2


def _np_split(key):
    b1, b2 = _tf2x32(key[0], key[1], np.uint32([0, 0]), np.uint32([0, 1]))
    return (np.array([b1[0], b2[0]], np.uint32), np.array([b1[1], b2[1]], np.uint32))


def _np_uniform01(key, n):
    bits = _np_random_bits(key, n)
    fb = (bits >> np.uint32(9)) | np.uint32(0x3F800000)
    return fb.view(np.float32) - np.float32(1.0)


def _np_permutation(key, n):
    x = np.arange(n, dtype=np.int64)
    exponent = 3
    num_rounds = int(np.ceil(exponent * np.log(max(1, n)) / np.log(np.iinfo(np.uint32).max)))
    for _ in range(num_rounds):
        key, subkey = _np_split(key)
        sort_keys = _np_random_bits(subkey, n)
        x = x[np.argsort(sort_keys, kind="stable")]
    return x


def _pack(a, k):
    n = _NW * k * _CBL
    assert len(a) <= n, (len(a), n)
    a2 = np.concatenate([a, np.full(n - len(a), a[0], a.dtype)])
    return np.ascontiguousarray(a2.reshape(_NW, k, _CBL).astype(np.int32))


def _build_consts():
    key = np.array([0, 42], np.uint32)
    dec = _np_uniform01(_np_fold_in(key, 0), _B) < 0.5
    perms = [np.arange(_B, dtype=np.int64)]
    for i in range(1, _N):
        perms.append(_np_permutation(_np_fold_in(key, i), _B))
    ar = np.arange(_B, dtype=np.int64)
    ndec = ~dec

    srcs = np.stack([np.where(dec if i else np.zeros(_B, bool), perms[i], ar)
                     for i in range(_N)], axis=1)
    etidx = np.ascontiguousarray(
        srcs.reshape(_NW, _KE2, _CBE, _N).transpose(0, 3, 1, 2).astype(np.int32))

    c_src = ar[ndec]
    c_dst = 5 * ar[ndec]
    q_src = np.concatenate([perms[j - 1][dec] for j in range(1, 5)])
    q_dst = np.concatenate([5 * ar[dec] + j for j in range(1, 5)])
    z_dst = np.concatenate([5 * ar[dec]] + [5 * ar[ndec] + j for j in range(1, 5)])

    return (etidx, _pack(c_src, _KC), _pack(c_dst, _KC),
            _pack(q_src, _KQ), _pack(q_dst, _KQ), _pack(z_dst, _KZ))


_CONSTS = _build_consts()


def _body(label_h, csrc_h, cdst_h, qsrc_h, qdst_h, zdst_h,
          out_l, csv, cdv, qsv, qdv, zdv, lb, zb,
          g0, g1, g2, g3, s0, s1, s2, s3, zsem):
    w = lax.axis_index("c") * _NS + lax.axis_index("s")
    gsems = (g0, g1, g2, g3)
    ssems = (s0, s1, s2, s3)

    pltpu.sync_copy(csrc_h.at[w], csv)
    pltpu.sync_copy(cdst_h.at[w], cdv)
    pltpu.sync_copy(qsrc_h.at[w], qsv)
    pltpu.sync_copy(qdst_h.at[w], qdv)
    pltpu.sync_copy(zdst_h.at[w], zdv)

    zv = jnp.zeros((16,), jnp.float32)

    def zrow(r, _):
        for t in range(62):
            zb[r, pl.ds(t * 16, 16)] = zv
        zb[r, pl.ds(984, 16)] = zv
        return _

    lax.fori_loop(0, _CBL, zrow, None)

    def zgroup(g, _):
        hs = [pltpu.async_copy(zb, out_l.at[zdv.at[g * 8 + t]], zsem)
              for t in range(8)]
        for h in hs:
            h.wait()
        return _

    lax.fori_loop(0, _KZ // 8, zgroup, None)

    def cbody(i, _):
        hs, ss = [], []
        for t in range(4):
            k = i * 4 + t
            hs.append(pltpu.async_copy(label_h.at[csv.at[k]], lb.at[t], gsems[t]))
        for t in range(4):
            k = i * 4 + t
            hs[t].wait()
            ss.append(pltpu.async_copy(lb.at[t], out_l.at[cdv.at[k]], ssems[t]))
        for s in ss:
            s.wait()
        return _

    lax.fori_loop(0, _KC // 4, cbody, None)

    qs = jnp.full((16,), 0.25, jnp.float32)
    qt = jnp.where(lax.iota(jnp.int32, 16) < 8, 1.0, 0.25).astype(jnp.float32)

    def qbody(i, _):
        hs, ss = [], []
        for t in range(4):
            k = i * 4 + t
            hs.append(pltpu.async_copy(label_h.at[qsv.at[k]], lb.at[t], gsems[t]))
        for t in range(4):
            k = i * 4 + t
            hs[t].wait()

            def srow(r, _2, t=t):
                for c in range(62):
                    lb[t, r, pl.ds(c * 16, 16)] = lb[t, r, pl.ds(c * 16, 16)] * qs
                lb[t, r, pl.ds(984, 16)] = lb[t, r, pl.ds(984, 16)] * qt
                return _2

            lax.fori_loop(0, _CBL, srow, None)
            ss.append(pltpu.async_copy(lb.at[t], out_l.at[qdv.at[k]], ssems[t]))
        for s in ss:
            s.wait()
        return _

    lax.fori_loop(0, _KQ // 4, qbody, None)


@functools.cache
def _sc_call():
    return pl.kernel(
        _body,
        out_type=jax.ShapeDtypeStruct((_B * 5, _L), jnp.float32),
        mesh=plsc.VectorSubcoreMesh(
            core_axis_name="c", subcore_axis_name="s", num_cores=_NC, num_subcores=_NS
        ),
        compiler_params=pltpu.CompilerParams(use_tc_tiling_on_sc=False),
        scratch_types=[
            pltpu.VMEM((_KC, _CBL), jnp.int32),
            pltpu.VMEM((_KC, _CBL), jnp.int32),
            pltpu.VMEM((_KQ, _CBL), jnp.int32),
            pltpu.VMEM((_KQ, _CBL), jnp.int32),
            pltpu.VMEM((_KZ, _CBL), jnp.int32),
            pltpu.VMEM((4, _CBL, _L), jnp.float32),
            pltpu.VMEM((_CBL, _L), jnp.float32),
        ] + [pltpu.SemaphoreType.DMA] * 9,
    )


def _ebody(embed_h, etidx_h, out_e, ejv, eb, eg0, eg1, es0, es1):
    w = lax.axis_index("c") * _NS + lax.axis_index("s")
    egsems = (eg0, eg1)
    essems = (es0, es1)
    pltpu.sync_copy(etidx_h.at[w], ejv)
    for j in range(_N):
        col = pl.ds(j * _SUB, _SUB)

        def body(i, _, j=j, col=col):
            hs, ss = [], []
            for t in range(2):
                k = i * 2 + t
                hs.append(pltpu.async_copy(
                    embed_h.at[ejv.at[j, k], col], eb.at[t], egsems[t]))
            for t in range(2):
                k = i * 2 + t
                hs[t].wait()
                base = w * (_B // _NW) + k * _CBE
                ss.append(pltpu.async_copy(
                    eb.at[t], out_e.at[pl.ds(base, _CBE), col], essems[t]))
            for s in ss:
                s.wait()
            return _

        lax.fori_loop(0, _KE2 // 2, body, None)


@functools.cache
def _sc_embed_call():
    return pl.kernel(
        _ebody,
        out_type=jax.ShapeDtypeStruct((_B, _D), jnp.float32),
        mesh=plsc.VectorSubcoreMesh(
            core_axis_name="c", subcore_axis_name="s", num_cores=_NC, num_subcores=_NS
        ),
        compiler_params=pltpu.CompilerParams(use_tc_tiling_on_sc=True),
        scratch_types=[
            pltpu.VMEM((_N, _KE2, _CBE), jnp.int32),
            pltpu.VMEM((2, _CBE, _SUB), jnp.float32),
        ] + [pltpu.SemaphoreType.DMA] * 4,
    )


def kernel(embed, onehot_label):
    etidx, *lconsts = [jnp.asarray(c) for c in _CONSTS]
    out_e = _sc_embed_call()(embed, etidx)
    out_l = _sc_call()(onehot_label, *lconsts)
    return out_e, out_l.reshape(_B, 5 * _L)
